# baseline (speedup 1.0000x reference)
import functools

import jax
import jax.numpy as jnp
import numpy as np
from jax import lax
from jax.experimental import pallas as pl
from jax.experimental.pallas import tpu as pltpu
from jax.experimental.pallas import tpu_sc as plsc

_LEVELS = np.array([8, 5, 5, 5])
_G = 4
_NQ = 8
_L = 4
_DIM = 768
_DG = _DIM // _G
_GL = _G * _L
_TB = 1024
_NTOK = 16 * 1024
_NBINS = 1024
_NROW = _G * _NQ


def _fsq_body(x_ref, w2t_ref, wout2_ref, bin_ref, bout_ref, scale_ref,
              bc_ref, idx_ref, q_ref, loss_ref, loss_acc):
    i = pl.program_id(0)
    nsteps = pl.num_programs(0)

    @pl.when(i == 0)
    def _init():
        loss_acc[0] = 0.0

    xblk = x_ref[...]
    z_tok = jax.lax.dot_general(
        xblk, w2t_ref[...], (((1,), (0,)), ((), ())),
        preferred_element_type=jnp.float32)
    z = z_tok.T + bin_ref[...]

    half_l = bc_ref[:, 0:1]
    offset = bc_ref[:, 1:2]
    shift = bc_ref[:, 2:3]
    half_w = bc_ref[:, 3:4]

    resid = z
    qout = jnp.zeros_like(z)
    for q in range(_NQ):
        scale = scale_ref[:, q:q + 1]
        zq = jnp.tanh(resid / scale + shift) * half_l - offset
        codes = jnp.round(zq)
        quant = (codes / half_w) * scale
        resid = resid - quant
        qout = qout + quant
        d = (codes + half_w).reshape(_G, _L, _TB)
        p = (d[:, 0, :] + 8.0 * d[:, 1, :]).astype(jnp.int32)
        h = (d[:, 2, :] + 5.0 * d[:, 3, :]).astype(jnp.int32)
        idx_ref[:, q, :] = p + 40 * h

    out = jax.lax.dot_general(
        qout, wout2_ref[...], (((0,), (0,)), ((), ())),
        preferred_element_type=jnp.float32) + bout_ref[...]
    q_ref[...] = out
    diff = out - xblk
    loss_acc[0] += jnp.sum(diff * diff)

    @pl.when(i == nsteps - 1)
    def _fin():
        loss_ref[...] = jnp.full((1, 1), loss_acc[0] / float(nsteps * _TB * _DIM))


_SC_CHUNK = _NTOK // 16


def _hist_body(idx_hbm, cnt_hbm, idx_v, hist_v, fold_v):
    wid = lax.axis_index("s") * 2 + lax.axis_index("c")
    pltpu.sync_copy(idx_hbm.at[wid], idx_v)
    zero16 = jnp.zeros((16,), jnp.float32)
    ones16 = jnp.ones((16,), jnp.float32)
    lanes = lax.broadcasted_iota(jnp.int32, (16,), 0)

    def _zero(k, _):
        for r in range(16):
            hist_v[r, pl.ds(k * 16, 16)] = zero16
        return 0
    lax.fori_loop(0, _NBINS // 16, _zero, 0, unroll=2)

    def _scat(k, _):
        iv = idx_v[pl.ds(k * 16, 16)]
        plsc.addupdate_scatter(hist_v, [lanes, iv], ones16)
        return 0
    lax.fori_loop(0, _NTOK // 16, _scat, 0, unroll=8)

    def _fold(c, _):
        acc = hist_v[0, pl.ds(c * 16, 16)]
        for r in range(1, 16):
            acc = acc + hist_v[r, pl.ds(c * 16, 16)]
        fold_v[pl.ds(c * 16, 16)] = acc
        return 0
    lax.fori_loop(0, _NBINS // 16, _fold, 0, unroll=2)
    pltpu.sync_copy(fold_v, cnt_hbm.at[wid])


_hist_call = pl.kernel(
    _hist_body,
    out_type=jax.ShapeDtypeStruct((_NROW, _NBINS), jnp.float32),
    mesh=plsc.VectorSubcoreMesh(core_axis_name="c", subcore_axis_name="s"),
    scratch_types=[
        pltpu.VMEM((_NTOK,), jnp.int32),
        pltpu.VMEM((16, _NBINS), jnp.float32),
        pltpu.VMEM((_NBINS,), jnp.float32),
    ],
    compiler_params=pltpu.CompilerParams(
        use_tc_tiling_on_sc=False, needs_layout_passes=False),
)


def _ent_body(cnt_ref, met_ref):
    probs = cnt_ref[...] * (1.0 / float(_NTOK))
    plogp = jnp.where(probs > 0, probs * jnp.log(probs + 1e-10), 0.0)
    ent = -jnp.sum(plogp, axis=1, keepdims=True)
    met_ref[...] = jnp.exp(ent)


@jax.jit
def kernel(x, w_in, b_in, w_out, b_out):
    B, T, D = x.shape
    ntok = B * T
    nsteps = ntok // _TB
    xf = x.reshape(ntok, D)

    w2t = jax.scipy.linalg.block_diag(*[w_in[g] for g in range(_G)])
    wout2 = jax.scipy.linalg.block_diag(*[w_out[g] for g in range(_G)])
    bin_c = b_in.reshape(_GL, 1)
    bout_r = b_out.reshape(1, D)

    levels = jnp.tile(jnp.asarray(_LEVELS, jnp.float32), _G)
    eps = 1e-3
    half_l = (levels - 1.0) * (1.0 - eps) / 2.0
    offset = jnp.tile(jnp.where(jnp.asarray(_LEVELS % 2 == 0), 0.5, 0.0), _G)
    shift = jnp.arctanh(offset / half_l)
    half_w = jnp.tile(jnp.asarray(_LEVELS // 2, jnp.float32), _G)
    qs = jnp.arange(_NQ, dtype=jnp.float32)
    scales = (levels - 1.0)[:, None] ** (-qs[None, :])
    bconsts = jnp.stack([half_l, offset, shift, half_w], axis=1)

    const_spec = pl.BlockSpec(index_map=lambda i: (0, 0))
    idx_t, qf, loss = pl.pallas_call(
        _fsq_body,
        grid=(nsteps,),
        in_specs=[
            pl.BlockSpec((_TB, D), lambda i: (i, 0)),
            const_spec, const_spec, const_spec, const_spec, const_spec,
            const_spec,
        ],
        out_specs=[
            pl.BlockSpec((_G, _NQ, _TB), lambda i: (0, 0, i)),
            pl.BlockSpec((_TB, D), lambda i: (i, 0)),
            pl.BlockSpec((1, 1), lambda i: (0, 0)),
        ],
        out_shape=[
            jax.ShapeDtypeStruct((_G, _NQ, ntok), jnp.int32),
            jax.ShapeDtypeStruct((ntok, D), jnp.float32),
            jax.ShapeDtypeStruct((1, 1), jnp.float32),
        ],
        scratch_shapes=[
            pltpu.SMEM((1,), jnp.float32),
        ],
        compiler_params=pltpu.CompilerParams(
            dimension_semantics=("arbitrary",)),
    )(xf, w2t, wout2, bin_c, bout_r, scales, bconsts)

    counts = _hist_call(idx_t.reshape(_NROW, ntok))

    met32 = pl.pallas_call(
        _ent_body,
        out_shape=jax.ShapeDtypeStruct((_NROW, 1), jnp.float32),
    )(counts)

    all_indices = idx_t.transpose(0, 2, 1).reshape(_G, B, T, _NQ)
    quantized = qf.reshape(B, T, D)
    return (all_indices, quantized, loss.reshape(()), met32.reshape(_G, _NQ))


# speedup vs baseline: 1.0676x; 1.0676x over previous
import functools

import jax
import jax.numpy as jnp
import numpy as np
from jax import lax
from jax.experimental import pallas as pl
from jax.experimental.pallas import tpu as pltpu
from jax.experimental.pallas import tpu_sc as plsc

_LEVELS = np.array([8, 5, 5, 5])
_G = 4
_NQ = 8
_L = 4
_DIM = 768
_DG = _DIM // _G
_GL = _G * _L
_TB = 1024
_NTOK = 16 * 1024
_NBINS = 1024
_NROW = _G * _NQ


def _fsq_body(x_ref, w2t_ref, wout2_ref, bin_ref, bout_ref, isc_ref,
              qm_ref, bc_ref, ones_ref, idx_ref, q_ref, loss_ref, loss_acc):
    i = pl.program_id(0)
    nsteps = pl.num_programs(0)

    @pl.when(i == 0)
    def _init():
        loss_acc[...] = jnp.zeros_like(loss_acc)

    xblk = x_ref[...]
    z_tok = jax.lax.dot_general(
        xblk, w2t_ref[...], (((1,), (0,)), ((), ())),
        preferred_element_type=jnp.float32)
    z = z_tok.T + bin_ref[...]

    half_l = bc_ref[:, 0:1]
    offset = bc_ref[:, 1:2]
    shift = bc_ref[:, 2:3]
    half_w = bc_ref[:, 3:4]

    resid = z
    qout = jnp.zeros_like(z)
    for q in range(_NQ):
        inv_scale = isc_ref[:, q:q + 1]
        qmul = qm_ref[:, q:q + 1]
        zq = jnp.tanh(resid * inv_scale + shift) * half_l - offset
        codes = jnp.round(zq)
        quant = codes * qmul
        resid = resid - quant
        qout = qout + quant
        d = (codes + half_w).reshape(_G, _L, _TB)
        p = (d[:, 0, :] + 8.0 * d[:, 1, :]).astype(jnp.int32)
        h = (d[:, 2, :] + 5.0 * d[:, 3, :]).astype(jnp.int32)
        idx_ref[:, q, :] = p + 40 * h

    out = jax.lax.dot_general(
        qout, wout2_ref[...], (((0,), (0,)), ((), ())),
        preferred_element_type=jnp.float32) + bout_ref[...]
    q_ref[...] = out
    diff = out - xblk
    psum = jax.lax.dot_general(
        ones_ref[...], diff * diff, (((1,), (0,)), ((), ())),
        preferred_element_type=jnp.float32)
    loss_acc[...] += psum

    @pl.when(i == nsteps - 1)
    def _fin():
        loss_ref[...] = jnp.sum(loss_acc[...], axis=1, keepdims=True) * (
            1.0 / float(nsteps * _TB * _DIM))


def _sc_log(v):
    bits = lax.bitcast_convert_type(v, jnp.int32)
    e = ((bits >> 23) & 0xFF) - 127
    m = lax.bitcast_convert_type((bits & 0x7FFFFF) | 0x3F800000, jnp.float32)
    s = (m - 1.0) / (m + 1.0)
    s2 = s * s
    ln_m = 2.0 * s * (1.0 + s2 * (1.0 / 3.0 + s2 * (0.2 + s2 * (1.0 / 7.0))))
    return e.astype(jnp.float32) * 0.6931471805599453 + ln_m


def _hist_body(idx_hbm, met_hbm, idx_v, hist_v, met_v):
    wid = lax.axis_index("s") * 2 + lax.axis_index("c")
    pltpu.sync_copy(idx_hbm.at[wid], idx_v)
    zero16 = jnp.zeros((16,), jnp.float32)
    ones16 = jnp.ones((16,), jnp.float32)
    lanes = lax.broadcasted_iota(jnp.int32, (16,), 0)

    def _zero(k, _):
        for r in range(16):
            hist_v[r, pl.ds(k * 16, 16)] = zero16
        return 0
    lax.fori_loop(0, _NBINS // 16, _zero, 0, unroll=2)

    def _scat(k, _):
        iv = idx_v[pl.ds(k * 16, 16)]
        plsc.addupdate_scatter(hist_v, [lanes, iv], ones16)
        return 0
    lax.fori_loop(0, _NTOK // 16, _scat, 0, unroll=16)

    inv_n = 1.0 / float(_NTOK)

    def _ent(c, acc):
        cnt = hist_v[0, pl.ds(c * 16, 16)]
        for r in range(1, 16):
            cnt = cnt + hist_v[r, pl.ds(c * 16, 16)]
        probs = cnt * inv_n
        plogp = jnp.where(probs > 0, probs * _sc_log(probs + 1e-10), 0.0)
        return acc - plogp
    ent16 = lax.fori_loop(0, _NBINS // 16, _ent, zero16, unroll=2)
    ent = jnp.sum(ent16)
    met_v[...] = jnp.exp(jnp.full((16,), ent, jnp.float32))
    pltpu.sync_copy(met_v, met_hbm.at[wid])


_hist_call = pl.kernel(
    _hist_body,
    out_type=jax.ShapeDtypeStruct((_NROW, 16), jnp.float32),
    mesh=plsc.VectorSubcoreMesh(core_axis_name="c", subcore_axis_name="s"),
    scratch_types=[
        pltpu.VMEM((_NTOK,), jnp.int32),
        pltpu.VMEM((16, _NBINS), jnp.float32),
        pltpu.VMEM((16,), jnp.float32),
    ],
    compiler_params=pltpu.CompilerParams(
        use_tc_tiling_on_sc=False, needs_layout_passes=False),
)


@jax.jit
def kernel(x, w_in, b_in, w_out, b_out):
    B, T, D = x.shape
    ntok = B * T
    nsteps = ntok // _TB
    xf = x.reshape(ntok, D)

    w2t = jax.scipy.linalg.block_diag(*[w_in[g] for g in range(_G)])
    wout2 = jax.scipy.linalg.block_diag(*[w_out[g] for g in range(_G)])
    bin_c = b_in.reshape(_GL, 1)
    bout_r = b_out.reshape(1, D)

    levels = jnp.tile(jnp.asarray(_LEVELS, jnp.float32), _G)
    eps = 1e-3
    half_l = (levels - 1.0) * (1.0 - eps) / 2.0
    offset = jnp.tile(jnp.where(jnp.asarray(_LEVELS % 2 == 0), 0.5, 0.0), _G)
    shift = jnp.arctanh(offset / half_l)
    half_w = jnp.tile(jnp.asarray(_LEVELS // 2, jnp.float32), _G)
    qs = jnp.arange(_NQ, dtype=jnp.float32)
    inv_scales = (levels - 1.0)[:, None] ** qs[None, :]
    scales = (levels - 1.0)[:, None] ** (-qs[None, :])
    qmuls = scales / half_w[:, None]
    bconsts = jnp.stack([half_l, offset, shift, half_w], axis=1)
    ones_row = jnp.ones((1, _TB), jnp.float32)

    const_spec = pl.BlockSpec(index_map=lambda i: (0, 0))
    idx_t, qf, loss = pl.pallas_call(
        _fsq_body,
        grid=(nsteps,),
        in_specs=[
            pl.BlockSpec((_TB, D), lambda i: (i, 0)),
            const_spec, const_spec, const_spec, const_spec, const_spec,
            const_spec, const_spec, const_spec,
        ],
        out_specs=[
            pl.BlockSpec((_G, _NQ, _TB), lambda i: (0, 0, i)),
            pl.BlockSpec((_TB, D), lambda i: (i, 0)),
            pl.BlockSpec((1, 1), lambda i: (0, 0)),
        ],
        out_shape=[
            jax.ShapeDtypeStruct((_G, _NQ, ntok), jnp.int32),
            jax.ShapeDtypeStruct((ntok, D), jnp.float32),
            jax.ShapeDtypeStruct((1, 1), jnp.float32),
        ],
        scratch_shapes=[
            pltpu.VMEM((1, _DIM), jnp.float32),
        ],
        compiler_params=pltpu.CompilerParams(
            dimension_semantics=("arbitrary",)),
    )(xf, w2t, wout2, bin_c, bout_r, inv_scales, qmuls, bconsts, ones_row)

    met = _hist_call(idx_t.reshape(_NROW, ntok))

    all_indices = idx_t.transpose(0, 2, 1).reshape(_G, B, T, _NQ)
    quantized = qf.reshape(B, T, D)
    return (all_indices, quantized, loss.reshape(()), met[:, 0].reshape(_G, _NQ))


# speedup vs baseline: 1.4935x; 1.3989x over previous
import functools

import jax
import jax.numpy as jnp
import numpy as np
from jax.experimental import pallas as pl
from jax.experimental.pallas import tpu as pltpu

_LEVELS = np.array([8, 5, 5, 5])
_G = 4
_NQ = 8
_L = 4
_DIM = 768
_DG = _DIM // _G
_GL = _G * _L
_TB = 1024


def _fsq_body(x_ref, w2t_ref, wout2_ref, bin_ref, bout_ref, isc_ref,
              qm_ref, bc_ref, ones_ref, idx_ref, q_ref, loss_ref, met_ref,
              hist_acc, loss_acc):
    i = pl.program_id(0)
    nsteps = pl.num_programs(0)

    @pl.when(i == 0)
    def _init():
        hist_acc[...] = jnp.zeros_like(hist_acc)
        loss_acc[...] = jnp.zeros_like(loss_acc)

    xblk = x_ref[...]
    z_tok = jax.lax.dot_general(
        xblk, w2t_ref[...], (((1,), (0,)), ((), ())),
        preferred_element_type=jnp.float32)
    z = z_tok.T + bin_ref[...]

    half_l = bc_ref[:, 0:1]
    offset = bc_ref[:, 1:2]
    shift = bc_ref[:, 2:3]
    half_w = bc_ref[:, 3:4]

    iota40 = jax.lax.broadcasted_iota(jnp.int32, (1, 40, 1), 1)
    iota25 = jax.lax.broadcasted_iota(jnp.int32, (1, 25, 1), 1)

    resid = z
    qout = jnp.zeros_like(z)
    hists = []
    for q in range(_NQ):
        inv_scale = isc_ref[:, q:q + 1]
        qmul = qm_ref[:, q:q + 1]
        zq = jnp.tanh(resid * inv_scale + shift) * half_l - offset
        codes = jnp.round(zq)
        quant = codes * qmul
        resid = resid - quant
        qout = qout + quant
        d = (codes + half_w).reshape(_G, _L, _TB)
        p = (d[:, 0, :] + 8.0 * d[:, 1, :]).astype(jnp.int32)
        h = (d[:, 2, :] + 5.0 * d[:, 3, :]).astype(jnp.int32)
        idx_ref[:, q, :] = p + 40 * h
        u = (p[:, None, :] == iota40).astype(jnp.bfloat16)
        v = (h[:, None, :] == iota25).astype(jnp.bfloat16)
        hq = jax.lax.dot_general(
            v, u, (((2,), (2,)), ((0,), (0,))),
            preferred_element_type=jnp.float32)
        hists.append(hq)
    hist_acc[...] += jnp.stack(hists, axis=1)

    out = jax.lax.dot_general(
        qout, wout2_ref[...], (((0,), (0,)), ((), ())),
        preferred_element_type=jnp.float32) + bout_ref[...]
    q_ref[...] = out
    diff = out - xblk
    loss_acc[...] += jax.lax.dot_general(
        ones_ref[...], diff * diff, (((1,), (0,)), ((), ())),
        preferred_element_type=jnp.float32)

    @pl.when(i == nsteps - 1)
    def _fin():
        ntok = nsteps * _TB
        loss_ref[...] = jnp.sum(loss_acc[...], axis=1, keepdims=True) * (
            1.0 / float(ntok * _DIM))
        probs = hist_acc[...] * (1.0 / float(ntok))
        plogp = jnp.where(probs > 0, probs * jnp.log(probs + 1e-10), 0.0)
        ent = -jnp.sum(jnp.sum(plogp, axis=3), axis=2)
        met_ref[...] = jnp.exp(ent)


@jax.jit
def kernel(x, w_in, b_in, w_out, b_out):
    B, T, D = x.shape
    ntok = B * T
    nsteps = ntok // _TB
    xf = x.reshape(ntok, D)

    w2t = jax.scipy.linalg.block_diag(*[w_in[g] for g in range(_G)])
    wout2 = jax.scipy.linalg.block_diag(*[w_out[g] for g in range(_G)])
    bin_c = b_in.reshape(_GL, 1)
    bout_r = b_out.reshape(1, D)

    levels = jnp.tile(jnp.asarray(_LEVELS, jnp.float32), _G)
    eps = 1e-3
    half_l = (levels - 1.0) * (1.0 - eps) / 2.0
    offset = jnp.tile(jnp.where(jnp.asarray(_LEVELS % 2 == 0), 0.5, 0.0), _G)
    shift = jnp.arctanh(offset / half_l)
    half_w = jnp.tile(jnp.asarray(_LEVELS // 2, jnp.float32), _G)
    qs = jnp.arange(_NQ, dtype=jnp.float32)
    inv_scales = (levels - 1.0)[:, None] ** qs[None, :]
    scales = (levels - 1.0)[:, None] ** (-qs[None, :])
    qmuls = scales / half_w[:, None]
    bconsts = jnp.stack([half_l, offset, shift, half_w], axis=1)
    ones_row = jnp.ones((1, _TB), jnp.float32)

    const_spec = pl.BlockSpec(index_map=lambda i: (0, 0))
    idx_t, qf, loss, met = pl.pallas_call(
        _fsq_body,
        grid=(nsteps,),
        in_specs=[
            pl.BlockSpec((_TB, D), lambda i: (i, 0)),
            const_spec, const_spec, const_spec, const_spec, const_spec,
            const_spec, const_spec, const_spec,
        ],
        out_specs=[
            pl.BlockSpec((_G, _NQ, _TB), lambda i: (0, 0, i)),
            pl.BlockSpec((_TB, D), lambda i: (i, 0)),
            pl.BlockSpec((1, 1), lambda i: (0, 0)),
            pl.BlockSpec((_G, _NQ), lambda i: (0, 0)),
        ],
        out_shape=[
            jax.ShapeDtypeStruct((_G, _NQ, ntok), jnp.int32),
            jax.ShapeDtypeStruct((ntok, D), jnp.float32),
            jax.ShapeDtypeStruct((1, 1), jnp.float32),
            jax.ShapeDtypeStruct((_G, _NQ), jnp.float32),
        ],
        scratch_shapes=[
            pltpu.VMEM((_G, _NQ, 25, 40), jnp.float32),
            pltpu.VMEM((1, _DIM), jnp.float32),
        ],
        compiler_params=pltpu.CompilerParams(
            dimension_semantics=("arbitrary",)),
    )(xf, w2t, wout2, bin_c, bout_r, inv_scales, qmuls, bconsts, ones_row)

    all_indices = idx_t.transpose(0, 2, 1).reshape(_G, B, T, _NQ)
    quantized = qf.reshape(B, T, D)
    return (all_indices, quantized, loss.reshape(()), met)


# speedup vs baseline: 1.5129x; 1.0130x over previous
import functools

import jax
import jax.numpy as jnp
import numpy as np
from jax.experimental import pallas as pl
from jax.experimental.pallas import tpu as pltpu

_LEVELS = np.array([8, 5, 5, 5])
_G = 4
_NQ = 8
_L = 4
_DIM = 768
_DG = _DIM // _G
_GL = _G * _L
_TB = 1024


def _fsq_body(x_ref, w2t_ref, wout2_ref, bin_ref, bout_ref, isc_ref,
              qm_ref, bc_ref, ones_ref, idx_ref, q_ref, loss_ref, met_ref,
              hist_acc, loss_acc):
    i = pl.program_id(0)
    nsteps = pl.num_programs(0)

    @pl.when(i == 0)
    def _init():
        hist_acc[...] = jnp.zeros_like(hist_acc)
        loss_acc[...] = jnp.zeros_like(loss_acc)

    xblk = x_ref[...]
    z_tok = jnp.concatenate(
        [jax.lax.dot_general(
            xblk[:, g * _DG:(g + 1) * _DG],
            w2t_ref[g * _DG:(g + 1) * _DG, 4 * g:4 * (g + 1)],
            (((1,), (0,)), ((), ())),
            preferred_element_type=jnp.float32)
         for g in range(_G)], axis=1)
    z = z_tok.T + bin_ref[...]

    half_l = bc_ref[:, 0:1]
    offset = bc_ref[:, 1:2]
    shift = bc_ref[:, 2:3]
    half_w = bc_ref[:, 3:4]

    iota40 = jax.lax.broadcasted_iota(jnp.int32, (1, 40, 1), 1)
    iota25 = jax.lax.broadcasted_iota(jnp.int32, (1, 25, 1), 1)

    resid = z
    qout = jnp.zeros_like(z)
    hists = []
    for q in range(_NQ):
        inv_scale = isc_ref[:, q:q + 1]
        qmul = qm_ref[:, q:q + 1]
        zq = jnp.tanh(resid * inv_scale + shift) * half_l - offset
        codes = jnp.round(zq)
        quant = codes * qmul
        resid = resid - quant
        qout = qout + quant
        d = (codes + half_w).reshape(_G, _L, _TB)
        p = (d[:, 0, :] + 8.0 * d[:, 1, :]).astype(jnp.int32)
        h = (d[:, 2, :] + 5.0 * d[:, 3, :]).astype(jnp.int32)
        idx_ref[:, q, :] = p + 40 * h
        u = (p[:, None, :] == iota40).astype(jnp.bfloat16)
        v = (h[:, None, :] == iota25).astype(jnp.bfloat16)
        hq = jax.lax.dot_general(
            v, u, (((2,), (2,)), ((0,), (0,))),
            preferred_element_type=jnp.float32)
        hists.append(hq)
    hist_acc[...] += jnp.stack(hists, axis=1)

    out = jax.lax.dot_general(
        qout, wout2_ref[...], (((0,), (0,)), ((), ())),
        preferred_element_type=jnp.float32) + bout_ref[...]
    q_ref[...] = out
    diff = out - xblk
    loss_acc[...] += jax.lax.dot_general(
        ones_ref[...], diff * diff, (((1,), (0,)), ((), ())),
        preferred_element_type=jnp.float32)

    @pl.when(i == nsteps - 1)
    def _fin():
        ntok = nsteps * _TB
        loss_ref[...] = jnp.sum(loss_acc[...], axis=1, keepdims=True) * (
            1.0 / float(ntok * _DIM))
        probs = hist_acc[...] * (1.0 / float(ntok))
        plogp = jnp.where(probs > 0, probs * jnp.log(probs + 1e-10), 0.0)
        ent = -jnp.sum(jnp.sum(plogp, axis=3), axis=2)
        met_ref[...] = jnp.exp(ent)


@jax.jit
def kernel(x, w_in, b_in, w_out, b_out):
    B, T, D = x.shape
    ntok = B * T
    nsteps = ntok // _TB
    xf = x.reshape(ntok, D)

    w2t = jax.scipy.linalg.block_diag(*[w_in[g] for g in range(_G)])
    wout2 = jax.scipy.linalg.block_diag(*[w_out[g] for g in range(_G)])
    bin_c = b_in.reshape(_GL, 1)
    bout_r = b_out.reshape(1, D)

    levels = jnp.tile(jnp.asarray(_LEVELS, jnp.float32), _G)
    eps = 1e-3
    half_l = (levels - 1.0) * (1.0 - eps) / 2.0
    offset = jnp.tile(jnp.where(jnp.asarray(_LEVELS % 2 == 0), 0.5, 0.0), _G)
    shift = jnp.arctanh(offset / half_l)
    half_w = jnp.tile(jnp.asarray(_LEVELS // 2, jnp.float32), _G)
    qs = jnp.arange(_NQ, dtype=jnp.float32)
    inv_scales = (levels - 1.0)[:, None] ** qs[None, :]
    scales = (levels - 1.0)[:, None] ** (-qs[None, :])
    qmuls = scales / half_w[:, None]
    bconsts = jnp.stack([half_l, offset, shift, half_w], axis=1)
    ones_row = jnp.ones((1, _TB), jnp.float32)

    const_spec = pl.BlockSpec(index_map=lambda i: (0, 0))
    idx_t, qf, loss, met = pl.pallas_call(
        _fsq_body,
        grid=(nsteps,),
        in_specs=[
            pl.BlockSpec((_TB, D), lambda i: (i, 0)),
            const_spec, const_spec, const_spec, const_spec, const_spec,
            const_spec, const_spec, const_spec,
        ],
        out_specs=[
            pl.BlockSpec((_G, _NQ, _TB), lambda i: (0, 0, i)),
            pl.BlockSpec((_TB, D), lambda i: (i, 0)),
            pl.BlockSpec((1, 1), lambda i: (0, 0)),
            pl.BlockSpec((_G, _NQ), lambda i: (0, 0)),
        ],
        out_shape=[
            jax.ShapeDtypeStruct((_G, _NQ, ntok), jnp.int32),
            jax.ShapeDtypeStruct((ntok, D), jnp.float32),
            jax.ShapeDtypeStruct((1, 1), jnp.float32),
            jax.ShapeDtypeStruct((_G, _NQ), jnp.float32),
        ],
        scratch_shapes=[
            pltpu.VMEM((_G, _NQ, 25, 40), jnp.float32),
            pltpu.VMEM((1, _DIM), jnp.float32),
        ],
        compiler_params=pltpu.CompilerParams(
            dimension_semantics=("arbitrary",)),
    )(xf, w2t, wout2, bin_c, bout_r, inv_scales, qmuls, bconsts, ones_row)

    all_indices = idx_t.transpose(0, 2, 1).reshape(_G, B, T, _NQ)
    quantized = qf.reshape(B, T, D)
    return (all_indices, quantized, loss.reshape(()), met)


# speedup vs baseline: 1.5623x; 1.0327x over previous
import functools

import jax
import jax.numpy as jnp
import numpy as np
from jax.experimental import pallas as pl
from jax.experimental.pallas import tpu as pltpu

_LEVELS = np.array([8, 5, 5, 5])
_G = 4
_NQ = 8
_L = 4
_DIM = 768
_DG = _DIM // _G
_GL = _G * _L
_TB = 2048


def _fsq_body(x_ref, w2t_ref, wout2_ref, bin_ref, bout_ref, isc_ref,
              qm_ref, bc_ref, ones_ref, idx_ref, q_ref, loss_ref, met_ref,
              hist_acc, loss_acc):
    i = pl.program_id(0)
    nsteps = pl.num_programs(0)

    @pl.when(i == 0)
    def _init():
        hist_acc[...] = jnp.zeros_like(hist_acc)
        loss_acc[...] = jnp.zeros_like(loss_acc)

    xblk = x_ref[...]
    z_tok = jnp.concatenate(
        [jax.lax.dot_general(
            xblk[:, g * _DG:(g + 1) * _DG],
            w2t_ref[g * _DG:(g + 1) * _DG, 4 * g:4 * (g + 1)],
            (((1,), (0,)), ((), ())),
            preferred_element_type=jnp.float32)
         for g in range(_G)], axis=1)
    z = z_tok.T + bin_ref[...]

    half_l = bc_ref[:, 0:1]
    offset = bc_ref[:, 1:2]
    shift = bc_ref[:, 2:3]
    half_w = bc_ref[:, 3:4]

    iota40 = jax.lax.broadcasted_iota(jnp.int32, (1, 40, 1), 1)
    iota25 = jax.lax.broadcasted_iota(jnp.int32, (1, 25, 1), 1)

    resid = z
    qout = jnp.zeros_like(z)
    hists = []
    for q in range(_NQ):
        inv_scale = isc_ref[:, q:q + 1]
        qmul = qm_ref[:, q:q + 1]
        zq = jnp.tanh(resid * inv_scale + shift) * half_l - offset
        codes = jnp.round(zq)
        quant = codes * qmul
        resid = resid - quant
        qout = qout + quant
        d = (codes + half_w).reshape(_G, _L, _TB)
        p = (d[:, 0, :] + 8.0 * d[:, 1, :]).astype(jnp.int32)
        h = (d[:, 2, :] + 5.0 * d[:, 3, :]).astype(jnp.int32)
        idx_ref[:, q, :] = p + 40 * h
        u = (p[:, None, :] == iota40).astype(jnp.bfloat16)
        v = (h[:, None, :] == iota25).astype(jnp.bfloat16)
        hq = jax.lax.dot_general(
            v, u, (((2,), (2,)), ((0,), (0,))),
            preferred_element_type=jnp.float32)
        hists.append(hq)
    hist_acc[...] += jnp.stack(hists, axis=1)

    out = jax.lax.dot_general(
        qout, wout2_ref[...], (((0,), (0,)), ((), ())),
        preferred_element_type=jnp.float32) + bout_ref[...]
    q_ref[...] = out
    diff = out - xblk
    loss_acc[...] += jax.lax.dot_general(
        ones_ref[...], diff * diff, (((1,), (0,)), ((), ())),
        preferred_element_type=jnp.float32)

    @pl.when(i == nsteps - 1)
    def _fin():
        ntok = nsteps * _TB
        loss_ref[...] = jnp.sum(loss_acc[...], axis=1, keepdims=True) * (
            1.0 / float(ntok * _DIM))
        probs = hist_acc[...] * (1.0 / float(ntok))
        plogp = jnp.where(probs > 0, probs * jnp.log(probs + 1e-10), 0.0)
        ent = -jnp.sum(jnp.sum(plogp, axis=3), axis=2)
        met_ref[...] = jnp.exp(ent)


@jax.jit
def kernel(x, w_in, b_in, w_out, b_out):
    B, T, D = x.shape
    ntok = B * T
    nsteps = ntok // _TB
    xf = x.reshape(ntok, D)

    w2t = jax.scipy.linalg.block_diag(*[w_in[g] for g in range(_G)])
    wout2 = jax.scipy.linalg.block_diag(*[w_out[g] for g in range(_G)])
    bin_c = b_in.reshape(_GL, 1)
    bout_r = b_out.reshape(1, D)

    levels = jnp.tile(jnp.asarray(_LEVELS, jnp.float32), _G)
    eps = 1e-3
    half_l = (levels - 1.0) * (1.0 - eps) / 2.0
    offset = jnp.tile(jnp.where(jnp.asarray(_LEVELS % 2 == 0), 0.5, 0.0), _G)
    shift = jnp.arctanh(offset / half_l)
    half_w = jnp.tile(jnp.asarray(_LEVELS // 2, jnp.float32), _G)
    qs = jnp.arange(_NQ, dtype=jnp.float32)
    inv_scales = (levels - 1.0)[:, None] ** qs[None, :]
    scales = (levels - 1.0)[:, None] ** (-qs[None, :])
    qmuls = scales / half_w[:, None]
    bconsts = jnp.stack([half_l, offset, shift, half_w], axis=1)
    ones_row = jnp.ones((1, _TB), jnp.float32)

    const_spec = pl.BlockSpec(index_map=lambda i: (0, 0))
    idx_t, qf, loss, met = pl.pallas_call(
        _fsq_body,
        grid=(nsteps,),
        in_specs=[
            pl.BlockSpec((_TB, D), lambda i: (i, 0)),
            const_spec, const_spec, const_spec, const_spec, const_spec,
            const_spec, const_spec, const_spec,
        ],
        out_specs=[
            pl.BlockSpec((_G, _NQ, _TB), lambda i: (0, 0, i)),
            pl.BlockSpec((_TB, D), lambda i: (i, 0)),
            pl.BlockSpec((1, 1), lambda i: (0, 0)),
            pl.BlockSpec((_G, _NQ), lambda i: (0, 0)),
        ],
        out_shape=[
            jax.ShapeDtypeStruct((_G, _NQ, ntok), jnp.int32),
            jax.ShapeDtypeStruct((ntok, D), jnp.float32),
            jax.ShapeDtypeStruct((1, 1), jnp.float32),
            jax.ShapeDtypeStruct((_G, _NQ), jnp.float32),
        ],
        scratch_shapes=[
            pltpu.VMEM((_G, _NQ, 25, 40), jnp.float32),
            pltpu.VMEM((1, _DIM), jnp.float32),
        ],
        compiler_params=pltpu.CompilerParams(
            dimension_semantics=("arbitrary",)),
    )(xf, w2t, wout2, bin_c, bout_r, inv_scales, qmuls, bconsts, ones_row)

    all_indices = idx_t.transpose(0, 2, 1).reshape(_G, B, T, _NQ)
    quantized = qf.reshape(B, T, D)
    return (all_indices, quantized, loss.reshape(()), met)
